# baseline (device time: 233019 ns/iter reference)
import numpy as np
import jax
import jax.numpy as jnp
from jax import lax
from jax.experimental import pallas as pl
from jax.experimental.pallas import tpu as pltpu

N_DEV = 4
SQ = 1024
SKV = 1024
H_LOC = 8
DH = 128
DM = 1024
BLK = 64
SCALE = 0.08838834764831843

_PBLK = [b for r in range(3) for b in range(16) if b % 3 == r]
_PERM = np.concatenate([np.arange(BLK) + BLK * b for b in _PBLK])
_INV = np.argsort(_PERM)
NA = 6 * BLK
NBC = 5 * BLK


def kernel(x, Wq, K_ext, V_ext, Wo):
    my = lax.axis_index("i")
    K_sl = lax.dynamic_slice_in_dim(K_ext, my * H_LOC, H_LOC, axis=2)
    V_sl = lax.dynamic_slice_in_dim(V_ext, my * H_LOC, H_LOC, axis=2)
    K_sl = K_sl[:, _PERM].transpose(0, 2, 1, 3).astype(jnp.bfloat16)
    V_sl = V_sl[:, _PERM].transpose(0, 2, 1, 3).astype(jnp.bfloat16)
    x_b = x[:, _PERM, :].astype(jnp.bfloat16)
    Wq_b = Wq.astype(jnp.bfloat16).reshape(DM, H_LOC, DH).transpose(1, 0, 2)
    Wo_b = Wo.astype(jnp.bfloat16).reshape(H_LOC, DH, DM)

    def body(x_ref, wq_ref, k_ref, v_ref, wo_ref, out_ref,
             xbuf, arecv, asend, xs, xr, as_, ar):
        my_pos = lax.axis_index("i")
        left = (my_pos + N_DEV - 1) % N_DEV
        right = (my_pos + 1) % N_DEV

        def partial_for(x_val, b):

            def head_body(h, acc):
                k = k_ref[b, h]
                v = v_ref[b, h]
                q = (lax.dot_general(
                    x_val, wq_ref[h], (((1,), (0,)), ((), ())),
                    preferred_element_type=jnp.float32,
                ) * SCALE).astype(jnp.bfloat16)

                cdims = (((1,), (1,)), ((), ()))
                s_a = lax.dot_general(
                    q[0:NA], k[0:NA], cdims,
                    preferred_element_type=jnp.float32,
                )
                w_a = jnp.exp(s_a)
                d_a = jnp.sum(w_a, axis=1, keepdims=True)
                ctx_a = lax.dot_general(
                    w_a.astype(jnp.bfloat16), v[0:NA],
                    (((1,), (0,)), ((), ())),
                    preferred_element_type=jnp.float32,
                ) / d_a

                k_b = jnp.concatenate([k[0:BLK], k[NA + NBC:]], axis=0)
                v_b = jnp.concatenate([v[0:BLK], v[NA + NBC:]], axis=0)
                s_b = lax.dot_general(
                    q[NA:NA + NBC], k_b, cdims,
                    preferred_element_type=jnp.float32,
                )
                w_b = jnp.exp(s_b)
                d_b = jnp.sum(w_b, axis=1, keepdims=True)
                ctx_b = lax.dot_general(
                    w_b.astype(jnp.bfloat16), v_b,
                    (((1,), (0,)), ((), ())),
                    preferred_element_type=jnp.float32,
                )

                k_c = jnp.concatenate([k[0:BLK], k[NA:NA + NBC]], axis=0)
                v_c = jnp.concatenate([v[0:BLK], v[NA:NA + NBC]], axis=0)
                s_c = lax.dot_general(
                    q[NA + NBC:], k_c, cdims,
                    preferred_element_type=jnp.float32,
                )
                w_c = jnp.exp(s_c)
                d_c = jnp.sum(w_c, axis=1, keepdims=True)
                ctx_c = lax.dot_general(
                    w_c.astype(jnp.bfloat16), v_c,
                    (((1,), (0,)), ((), ())),
                    preferred_element_type=jnp.float32,
                )

                q_d = q[NA:].reshape(10, BLK, DH)
                k_d = k[NA:].reshape(10, BLK, DH)
                v_d = v[NA:].reshape(10, BLK, DH)
                s_d = lax.dot_general(
                    q_d, k_d, (((2,), (2,)), ((0,), (0,))),
                    preferred_element_type=jnp.float32,
                )
                w_d = jnp.exp(s_d)
                d_d = jnp.sum(w_d, axis=2, keepdims=True)
                ctx_d = lax.dot_general(
                    w_d.astype(jnp.bfloat16), v_d,
                    (((2,), (1,)), ((0,), (0,))),
                    preferred_element_type=jnp.float32,
                ).reshape(2 * NBC, DH)
                d_d = d_d.reshape(2 * NBC, 1)

                ctx = jnp.concatenate([
                    ctx_a,
                    (ctx_b + ctx_d[0:NBC]) / (d_b + d_d[0:NBC]),
                    (ctx_c + ctx_d[NBC:]) / (d_c + d_d[NBC:]),
                ], axis=0)
                return acc + lax.dot_general(
                    ctx.astype(jnp.bfloat16), wo_ref[h],
                    (((1,), (0,)), ((), ())),
                    preferred_element_type=jnp.float32,
                )

            return lax.fori_loop(
                0, H_LOC, head_body, jnp.zeros((SQ, DM), jnp.float32)
            )

        def xcopy(src, slot):
            return pltpu.make_async_remote_copy(
                src_ref=src, dst_ref=xbuf.at[slot],
                send_sem=xs.at[slot], recv_sem=xr.at[slot],
                device_id=(right,), device_id_type=pl.DeviceIdType.MESH,
            )

        def acopy(slot):
            return pltpu.make_async_remote_copy(
                src_ref=asend, dst_ref=arecv.at[slot],
                send_sem=as_.at[slot], recv_sem=ar.at[slot],
                device_id=(right,), device_id_type=pl.DeviceIdType.MESH,
            )

        barrier = pltpu.get_barrier_semaphore()
        for nbr in (left, right):
            pl.semaphore_signal(
                barrier, inc=1,
                device_id=(nbr,), device_id_type=pl.DeviceIdType.MESH,
            )
        pl.semaphore_wait(barrier, 2)

        cx0 = xcopy(x_ref.at[0], 0)
        cx0.start()
        p_own = partial_for(x_ref[0], my_pos)

        cx0.wait()
        cx1 = xcopy(xbuf.at[0], 1)
        cx1.start()
        p = partial_for(xbuf[0], (my_pos + 3) % N_DEV)
        asend[:, :] = p.astype(jnp.bfloat16)
        ca0 = acopy(0)
        ca0.start()

        cx1.wait()
        cx2 = xcopy(xbuf.at[1], 2)
        cx2.start()
        p = partial_for(xbuf[1], (my_pos + 2) % N_DEV)
        ca0.wait()
        asend[:, :] = (arecv[0] + p).astype(jnp.bfloat16)
        ca1 = acopy(1)
        ca1.start()

        cx2.wait()
        p = partial_for(xbuf[2], (my_pos + 1) % N_DEV)
        ca1.wait()
        asend[:, :] = (arecv[1] + p).astype(jnp.bfloat16)
        ca2 = acopy(2)
        ca2.start()

        ca2.wait()
        out_ref[0, :, :] = arecv[2] + p_own

    out = pl.pallas_call(
        body,
        out_shape=jax.ShapeDtypeStruct((1, SQ, DM), jnp.float32),
        in_specs=[pl.BlockSpec(memory_space=pltpu.VMEM)] * 5,
        out_specs=pl.BlockSpec(memory_space=pltpu.VMEM),
        scratch_shapes=[
            pltpu.VMEM((3, SQ, DM), jnp.bfloat16),
            pltpu.VMEM((3, SQ, DM), jnp.bfloat16),
            pltpu.VMEM((SQ, DM), jnp.bfloat16),
            pltpu.SemaphoreType.DMA((3,)),
            pltpu.SemaphoreType.DMA((3,)),
            pltpu.SemaphoreType.DMA((3,)),
            pltpu.SemaphoreType.DMA((3,)),
        ],
        compiler_params=pltpu.CompilerParams(
            collective_id=0, vmem_limit_bytes=100 * 1024 * 1024,
        ),
    )(x_b, Wq_b, K_sl, V_sl, Wo_b)
    return out[:, _INV, :]
